# baseline (speedup 1.0000x reference)
import functools

import jax
import jax.numpy as jnp
from jax import lax
from jax.experimental import pallas as pl
from jax.experimental.pallas import tpu as pltpu
from jax.experimental.pallas import tpu_sc as plsc

_NC = 2
_NS = 16
_NW = _NC * _NS
_IW = 32
_K = 40
_C = _K * _IW


@functools.partial(jax.jit, static_argnames=("n_per_w", "d"))
def _sc_gather(idx, table, n_per_w, d):
    n = idx.shape[0]
    n_chunks = n_per_w // _C
    mesh = plsc.VectorSubcoreMesh(
        core_axis_name="c", subcore_axis_name="s",
        num_cores=_NC, num_subcores=_NS)

    @functools.partial(
        pl.kernel,
        out_type=jax.ShapeDtypeStruct((n // _IW, _IW, d), jnp.float32),
        mesh=mesh,
        scratch_types=[
            pltpu.VMEM((n_per_w,), jnp.int32),
            pltpu.VMEM((_K, _IW, d), jnp.float32),
            pltpu.VMEM((_K, _IW, d), jnp.float32),
            pltpu.SemaphoreType.DMA,
            pltpu.SemaphoreType.DMA,
        ],
        compiler_params=pltpu.CompilerParams(use_tc_tiling_on_sc=False),
    )
    def k(idx_hbm, table_hbm, out_hbm, idx_v, rows_a, rows_b, sem_a, sem_b):
        wid = lax.axis_index("s") * _NC + lax.axis_index("c")
        base = wid * n_per_w
        pltpu.sync_copy(idx_hbm.at[pl.ds(base, n_per_w)], idx_v)

        rows = (rows_a, rows_b)
        sems = (sem_a, sem_b)

        def gather(g, b):
            for j in range(_K):
                pltpu.async_copy(
                    table_hbm.at[idx_v.at[pl.ds(g * _C + j * _IW, _IW)]],
                    rows[b].at[j], sems[b])

        def drain(b):
            for j in range(_K):
                pltpu.make_async_copy(
                    table_hbm.at[idx_v.at[pl.ds(j * _IW, _IW)]],
                    rows[b].at[j], sems[b]).wait()

        gather(0, 0)

        def body(t, carry):
            for b in range(2):
                g = 2 * t + b

                @pl.when(g + 1 < n_chunks)
                def _():
                    gather(g + 1, 1 - b)

                drain(b)
                pltpu.sync_copy(
                    rows[b],
                    out_hbm.at[pl.ds(base // _IW + g * _K, _K)])
            return carry

        lax.fori_loop(0, n_chunks // 2, body, 0)

    return k(idx, table)


def kernel(x, table):
    b, h = x.shape
    d = table.shape[1]
    idx = x.reshape(-1).astype(jnp.int32)
    n = idx.shape[0]
    n_per_w = n // _NW
    assert n % _NW == 0 and n_per_w % (2 * _C) == 0
    out = _sc_gather(idx, table, n_per_w, d)
    return out.reshape(b, h, d)


# speedup vs baseline: 1.0005x; 1.0005x over previous
import functools

import jax
import jax.numpy as jnp
from jax import lax
from jax.experimental import pallas as pl
from jax.experimental.pallas import tpu as pltpu
from jax.experimental.pallas import tpu_sc as plsc

_NC = 2
_NS = 16
_NW = _NC * _NS
_IW = 64
_K = 25
_C = _K * _IW


@functools.partial(jax.jit, static_argnames=("n_per_w", "d"))
def _sc_gather(idx, table, n_per_w, d):
    n = idx.shape[0]
    n_chunks = n_per_w // _C
    mesh = plsc.VectorSubcoreMesh(
        core_axis_name="c", subcore_axis_name="s",
        num_cores=_NC, num_subcores=_NS)

    @functools.partial(
        pl.kernel,
        out_type=jax.ShapeDtypeStruct((n // _IW, _IW, d), jnp.float32),
        mesh=mesh,
        scratch_types=[
            pltpu.VMEM((n_per_w,), jnp.int32),
            pltpu.VMEM((_K, _IW, d), jnp.float32),
            pltpu.VMEM((_K, _IW, d), jnp.float32),
            pltpu.SemaphoreType.DMA,
            pltpu.SemaphoreType.DMA,
        ],
        compiler_params=pltpu.CompilerParams(use_tc_tiling_on_sc=False),
    )
    def k(idx_hbm, table_hbm, out_hbm, idx_v, rows_a, rows_b, sem_a, sem_b):
        wid = lax.axis_index("s") * _NC + lax.axis_index("c")
        base = wid * n_per_w
        pltpu.sync_copy(idx_hbm.at[pl.ds(base, n_per_w)], idx_v)

        rows = (rows_a, rows_b)
        sems = (sem_a, sem_b)

        def gather(g, b):
            for j in range(_K):
                pltpu.async_copy(
                    table_hbm.at[idx_v.at[pl.ds(g * _C + j * _IW, _IW)]],
                    rows[b].at[j], sems[b])

        def drain(b):
            for j in range(_K):
                pltpu.make_async_copy(
                    table_hbm.at[idx_v.at[pl.ds(j * _IW, _IW)]],
                    rows[b].at[j], sems[b]).wait()

        gather(0, 0)

        def body(t, carry):
            for b in range(2):
                g = 2 * t + b

                @pl.when(g + 1 < n_chunks)
                def _():
                    gather(g + 1, 1 - b)

                drain(b)
                pltpu.sync_copy(
                    rows[b],
                    out_hbm.at[pl.ds(base // _IW + g * _K, _K)])
            return carry

        lax.fori_loop(0, n_chunks // 2, body, 0)

    return k(idx, table)


def kernel(x, table):
    b, h = x.shape
    d = table.shape[1]
    idx = x.reshape(-1).astype(jnp.int32)
    n = idx.shape[0]
    n_per_w = n // _NW
    assert n % _NW == 0 and n_per_w % (2 * _C) == 0
    out = _sc_gather(idx, table, n_per_w, d)
    return out.reshape(b, h, d)


# speedup vs baseline: 1.0032x; 1.0027x over previous
import functools

import jax
import jax.numpy as jnp
from jax import lax
from jax.experimental import pallas as pl
from jax.experimental.pallas import tpu as pltpu
from jax.experimental.pallas import tpu_sc as plsc

_NC = 2
_NS = 16
_NW = _NC * _NS
_IW = 16
_K = 40
_C = _K * _IW


@functools.partial(jax.jit, static_argnames=("n_per_w", "d"))
def _sc_gather(idx, table, n_per_w, d):
    n = idx.shape[0]
    n_chunks = n_per_w // _C
    mesh = plsc.VectorSubcoreMesh(
        core_axis_name="c", subcore_axis_name="s",
        num_cores=_NC, num_subcores=_NS)

    @functools.partial(
        pl.kernel,
        out_type=jax.ShapeDtypeStruct((n // _IW, _IW, d), jnp.float32),
        mesh=mesh,
        scratch_types=[
            pltpu.VMEM((n_per_w,), jnp.int32),
            pltpu.VMEM((_K, _IW, d), jnp.float32),
            pltpu.VMEM((_K, _IW, d), jnp.float32),
            pltpu.SemaphoreType.DMA,
            pltpu.SemaphoreType.DMA,
        ],
        compiler_params=pltpu.CompilerParams(use_tc_tiling_on_sc=False),
    )
    def k(idx_hbm, table_hbm, out_hbm, idx_v, rows_a, rows_b, sem_a, sem_b):
        wid = lax.axis_index("s") * _NC + lax.axis_index("c")
        base = wid * n_per_w
        pltpu.sync_copy(idx_hbm.at[pl.ds(base, n_per_w)], idx_v)

        rows = (rows_a, rows_b)
        sems = (sem_a, sem_b)

        def gather(g, b):
            for j in range(_K):
                pltpu.async_copy(
                    table_hbm.at[idx_v.at[pl.ds(g * _C + j * _IW, _IW)]],
                    rows[b].at[j], sems[b])

        def drain(b):
            for j in range(_K):
                pltpu.make_async_copy(
                    table_hbm.at[idx_v.at[pl.ds(j * _IW, _IW)]],
                    rows[b].at[j], sems[b]).wait()

        gather(0, 0)

        def body(t, carry):
            for b in range(2):
                g = 2 * t + b

                @pl.when(g + 1 < n_chunks)
                def _():
                    gather(g + 1, 1 - b)

                drain(b)
                pltpu.sync_copy(
                    rows[b],
                    out_hbm.at[pl.ds(base // _IW + g * _K, _K)])
            return carry

        lax.fori_loop(0, n_chunks // 2, body, 0)

    return k(idx, table)


def kernel(x, table):
    b, h = x.shape
    d = table.shape[1]
    idx = x.reshape(-1).astype(jnp.int32)
    n = idx.shape[0]
    n_per_w = n // _NW
    assert n % _NW == 0 and n_per_w % (2 * _C) == 0
    out = _sc_gather(idx, table, n_per_w, d)
    return out.reshape(b, h, d)
